# baseline (speedup 1.0000x reference)
import functools

import jax
import jax.numpy as jnp
from jax import lax
from jax.experimental import pallas as pl
from jax.experimental.pallas import tpu as pltpu
from jax.experimental.pallas import tpu_sc as plsc

NC = 2
NS = 16
L = 16
NW = NC * NS
BATCH = 16384
D = 16
V = 1000000
RPB = 8
NBLK = V // RPB
BPW = BATCH // NW
CHUNK = 128
NCHUNK = BPW // CHUNK
GROUPS = CHUNK // L

TW = 4096
TGRID = (V + TW - 1) // TW


def _relayout_body(ut_ref, it_ref, uo_ref, io_ref):
    eye = jnp.eye(D, dtype=jnp.float32)
    for ref, out in ((ut_ref, uo_ref), (it_ref, io_ref)):
        y = lax.dot_general(
            ref[...], eye, (((0,), (0,)), ((), ())),
            precision=lax.Precision.HIGHEST,
            preferred_element_type=jnp.float32,
        )
        z = y.reshape(TW // RPB, RPB, D)
        for a in range(RPB):
            out[:, a * D:(a + 1) * D] = z[:, a, :]


def _relayout(uet, iet):
    return pl.pallas_call(
        _relayout_body,
        grid=(TGRID,),
        in_specs=[
            pl.BlockSpec((D, TW), lambda g: (0, g)),
            pl.BlockSpec((D, TW), lambda g: (0, g)),
        ],
        out_specs=[
            pl.BlockSpec((TW // RPB, 128), lambda g: (g, 0)),
            pl.BlockSpec((TW // RPB, 128), lambda g: (g, 0)),
        ],
        out_shape=[
            jax.ShapeDtypeStruct((NBLK, 128), jnp.float32),
            jax.ShapeDtypeStruct((NBLK, 128), jnp.float32),
        ],
    )(uet, iet)


def _bprmf_body(u_hbm, i_hbm, ue_hbm, ie_hbm, out_hbm,
                u_idx, i_idx, ub_idx, ib_idx, u_blk, i_blk, out_v,
                sem0, sem1):
    wid = lax.axis_index("s") * NC + lax.axis_index("c")

    pltpu.sync_copy(u_hbm.at[pl.ds(wid * NCHUNK, NCHUNK)], u_idx)
    pltpu.sync_copy(i_hbm.at[pl.ds(wid * NCHUNK, NCHUNK)], i_idx)

    for j in range(NCHUNK):
        for s in range(GROUPS):
            ub_idx[j, pl.ds(s * L, L)] = u_idx[j, pl.ds(s * L, L)] >> 3
            ib_idx[j, pl.ds(s * L, L)] = i_idx[j, pl.ds(s * L, L)] >> 3

    sems = [sem0, sem1]

    def fire(j):
        buf = j % 2
        return (
            pltpu.async_copy(ue_hbm.at[ub_idx.at[j]], u_blk.at[buf], sems[buf]),
            pltpu.async_copy(ie_hbm.at[ib_idx.at[j]], i_blk.at[buf], sems[buf]),
        )

    pending = fire(0)
    for j in range(NCHUNK):
        cu, ci = pending
        if j + 1 < NCHUNK:
            nxt = fire(j + 1)
        cu.wait()
        ci.wait()
        buf = j % 2
        lanes = lax.iota(jnp.int32, L)

        def group(s, _):
            uoffs = (u_idx[j, pl.ds(s * L, L)] & 7) << 4
            ioffs = (i_idx[j, pl.ds(s * L, L)] & 7) << 4
            acc = jnp.zeros((L,), jnp.float32)
            for k in range(L):
                ue = u_blk[buf, s * L + k, pl.ds(uoffs[k], L)]
                ie = i_blk[buf, s * L + k, pl.ds(ioffs[k], L)]
                acc = jnp.where(lanes == k, jnp.sum(ue * ie), acc)
            out_v[j, pl.ds(s * L, L)] = acc
            return _

        lax.fori_loop(0, GROUPS, group, None)
        if j + 1 < NCHUNK:
            pending = nxt

    pltpu.sync_copy(out_v, out_hbm.at[pl.ds(wid * NCHUNK, NCHUNK)])


@jax.jit
def kernel(u, i, user_emb, item_emb):
    mesh = plsc.VectorSubcoreMesh(core_axis_name="c", subcore_axis_name="s")
    f = pl.kernel(
        _bprmf_body,
        out_type=jax.ShapeDtypeStruct((NW * NCHUNK, CHUNK), jnp.float32),
        mesh=mesh,
        compiler_params=pltpu.CompilerParams(
            needs_layout_passes=False, use_tc_tiling_on_sc=True),
        scratch_types=[
            pltpu.VMEM((NCHUNK, CHUNK), jnp.int32),
            pltpu.VMEM((NCHUNK, CHUNK), jnp.int32),
            pltpu.VMEM((NCHUNK, CHUNK), jnp.int32),
            pltpu.VMEM((NCHUNK, CHUNK), jnp.int32),
            pltpu.VMEM((2, CHUNK, 128), jnp.float32),
            pltpu.VMEM((2, CHUNK, 128), jnp.float32),
            pltpu.VMEM((NCHUNK, CHUNK), jnp.float32),
            pltpu.SemaphoreType.DMA,
            pltpu.SemaphoreType.DMA,
        ],
    )
    u2 = u.reshape(NW * NCHUNK, CHUNK)
    i2 = i.reshape(NW * NCHUNK, CHUNK)
    ue2, ie2 = _relayout(user_emb.T, item_emb.T)
    return f(u2, i2, ue2, ie2).reshape(BATCH)


# speedup vs baseline: 2.3964x; 2.3964x over previous
import functools

import jax
import jax.numpy as jnp
from jax import lax
from jax.experimental import pallas as pl
from jax.experimental.pallas import tpu as pltpu
from jax.experimental.pallas import tpu_sc as plsc

NC = 2
NS = 16
L = 16
NW = NC * NS
BATCH = 16384
D = 16
V = 1000000
UPG = 1024
NGRP = (V + UPG - 1) // UPG
NROW = NGRP * 128
BPW = BATCH // NW
CHUNK = 128
NCHUNK = BPW // CHUNK
GROUPS = CHUNK // L

TW = 8192
TGRID = (V + TW - 1) // TW


def _relayout_body(ut_ref, it_ref, uo_ref, io_ref):
    xu = ut_ref[...]
    xi = it_ref[...]
    for bg in range(TW // UPG):
        for x, out in ((xu, uo_ref), (xi, io_ref)):
            y = x[:, bg * UPG:(bg + 1) * UPG].T
            pieces = [y[b * 128:(b + 1) * 128, :] for b in range(8)]
            out[bg * 128:(bg + 1) * 128, :] = jnp.concatenate(pieces, axis=1)


def _relayout(uet, iet):
    return pl.pallas_call(
        _relayout_body,
        grid=(TGRID,),
        in_specs=[
            pl.BlockSpec((D, TW), lambda g: (0, g)),
            pl.BlockSpec((D, TW), lambda g: (0, g)),
        ],
        out_specs=[
            pl.BlockSpec((TW // UPG * 128, 128), lambda g: (g, 0)),
            pl.BlockSpec((TW // UPG * 128, 128), lambda g: (g, 0)),
        ],
        out_shape=[
            jax.ShapeDtypeStruct((NROW, 128), jnp.float32),
            jax.ShapeDtypeStruct((NROW, 128), jnp.float32),
        ],
    )(uet, iet)


def _bprmf_body(u_hbm, i_hbm, ue_hbm, ie_hbm, out_hbm,
                u_idx, i_idx, ub_idx, ib_idx, u_blk, i_blk, out_v,
                sem0, sem1):
    wid = lax.axis_index("s") * NC + lax.axis_index("c")

    pltpu.sync_copy(u_hbm.at[pl.ds(wid * NCHUNK, NCHUNK)], u_idx)
    pltpu.sync_copy(i_hbm.at[pl.ds(wid * NCHUNK, NCHUNK)], i_idx)

    for j in range(NCHUNK):
        for s in range(GROUPS):
            uv = u_idx[j, pl.ds(s * L, L)]
            iv = i_idx[j, pl.ds(s * L, L)]
            ub_idx[j, pl.ds(s * L, L)] = ((uv >> 10) << 7) | (uv & 127)
            ib_idx[j, pl.ds(s * L, L)] = ((iv >> 10) << 7) | (iv & 127)

    sems = [sem0, sem1]

    def fire(j):
        buf = j % 2
        return (
            pltpu.async_copy(ue_hbm.at[ub_idx.at[j]], u_blk.at[buf], sems[buf]),
            pltpu.async_copy(ie_hbm.at[ib_idx.at[j]], i_blk.at[buf], sems[buf]),
        )

    pending = fire(0)
    for j in range(NCHUNK):
        cu, ci = pending
        if j + 1 < NCHUNK:
            nxt = fire(j + 1)
        cu.wait()
        ci.wait()
        buf = j % 2
        lanes = lax.iota(jnp.int32, L)

        def group(s, _):
            uoffs = ((u_idx[j, pl.ds(s * L, L)] >> 7) & 7) << 4
            ioffs = ((i_idx[j, pl.ds(s * L, L)] >> 7) & 7) << 4
            acc = jnp.zeros((L,), jnp.float32)
            for k in range(L):
                ue = u_blk[buf, s * L + k, pl.ds(uoffs[k], L)]
                ie = i_blk[buf, s * L + k, pl.ds(ioffs[k], L)]
                acc = jnp.where(lanes == k, jnp.sum(ue * ie), acc)
            out_v[j, pl.ds(s * L, L)] = acc
            return _

        lax.fori_loop(0, GROUPS, group, None)
        if j + 1 < NCHUNK:
            pending = nxt

    pltpu.sync_copy(out_v, out_hbm.at[pl.ds(wid * NCHUNK, NCHUNK)])


@jax.jit
def kernel(u, i, user_emb, item_emb):
    mesh = plsc.VectorSubcoreMesh(core_axis_name="c", subcore_axis_name="s")
    f = pl.kernel(
        _bprmf_body,
        out_type=jax.ShapeDtypeStruct((NW * NCHUNK, CHUNK), jnp.float32),
        mesh=mesh,
        compiler_params=pltpu.CompilerParams(
            needs_layout_passes=False, use_tc_tiling_on_sc=True),
        scratch_types=[
            pltpu.VMEM((NCHUNK, CHUNK), jnp.int32),
            pltpu.VMEM((NCHUNK, CHUNK), jnp.int32),
            pltpu.VMEM((NCHUNK, CHUNK), jnp.int32),
            pltpu.VMEM((NCHUNK, CHUNK), jnp.int32),
            pltpu.VMEM((2, CHUNK, 128), jnp.float32),
            pltpu.VMEM((2, CHUNK, 128), jnp.float32),
            pltpu.VMEM((NCHUNK, CHUNK), jnp.float32),
            pltpu.SemaphoreType.DMA,
            pltpu.SemaphoreType.DMA,
        ],
    )
    u2 = u.reshape(NW * NCHUNK, CHUNK)
    i2 = i.reshape(NW * NCHUNK, CHUNK)
    ue2, ie2 = _relayout(user_emb.T, item_emb.T)
    return f(u2, i2, ue2, ie2).reshape(BATCH)
